# Initial kernel scaffold; baseline (speedup 1.0000x reference)
import jax
import jax.numpy as jnp
from jax.experimental import pallas as pl


def kernel(x, bin_logits, embed_table, W, b):
    raise NotImplementedError("write your pallas kernel here")


# speedup vs baseline: 91.3995x; 91.3995x over previous
import functools

import jax
import jax.numpy as jnp
import numpy as np
from jax.experimental import pallas as pl
from jax.experimental.pallas import tpu as pltpu


def _minmax_body(x_ref, min_ref, max_ref, mn_s, mx_s, *, nsteps):
    i = pl.program_id(0)
    bmin = jnp.min(x_ref[...], axis=0, keepdims=True)
    bmax = jnp.max(x_ref[...], axis=0, keepdims=True)

    @pl.when(i == 0)
    def _():
        mn_s[...] = bmin
        mx_s[...] = bmax

    @pl.when(i > 0)
    def _():
        mn_s[...] = jnp.minimum(mn_s[...], bmin)
        mx_s[...] = jnp.maximum(mx_s[...], bmax)

    @pl.when(i == nsteps - 1)
    def _():
        min_ref[...] = mn_s[...]
        max_ref[...] = mx_s[...]


def _bucket_body(x_ref, mn_ref, mx_ref, bins_ref, td_ref, b2_ref, o_ref):
    mn = mn_ref[...]
    inv = 1.0 / (mx_ref[...] - mn + 1e-6)
    xn = (x_ref[...] - mn) * inv
    binsr = bins_ref[...]
    td = td_ref[...]
    nfeat = xn.shape[1]
    acc = None
    for f in range(nfeat):
        col = xn[:, f:f + 1]
        c = (col <= binsr).astype(jnp.float32)
        p = jnp.dot(c, td, preferred_element_type=jnp.float32,
                    precision=jax.lax.Precision.HIGHEST)
        acc = p if acc is None else acc + p
    o_ref[...] = acc + b2_ref[...]


def kernel(x, bin_logits, embed_table, W, b):
    N, F = x.shape
    NB = bin_logits.shape[0]
    D = W.shape[0]

    probs = jax.nn.softmax(bin_logits)
    bins = jnp.cumsum(probs)
    bins_ext = bins.at[NB - 1].set(jnp.inf).reshape(1, NB)
    table2 = embed_table @ W.T
    tdiff = jnp.concatenate([table2[:-1] - table2[1:], table2[-1:]], axis=0)
    b2 = (float(F) * b).reshape(1, D)

    R1 = 2000
    n1 = N // R1
    xmin, xmax = pl.pallas_call(
        functools.partial(_minmax_body, nsteps=n1),
        grid=(n1,),
        in_specs=[pl.BlockSpec((R1, F), lambda i: (i, 0))],
        out_specs=[pl.BlockSpec((1, F), lambda i: (0, 0)),
                   pl.BlockSpec((1, F), lambda i: (0, 0))],
        out_shape=[jax.ShapeDtypeStruct((1, F), jnp.float32),
                   jax.ShapeDtypeStruct((1, F), jnp.float32)],
        scratch_shapes=[pltpu.VMEM((1, F), jnp.float32),
                        pltpu.VMEM((1, F), jnp.float32)],
        compiler_params=pltpu.CompilerParams(
            dimension_semantics=("arbitrary",)),
    )(x)

    R2 = 1000
    n2 = N // R2
    out = pl.pallas_call(
        _bucket_body,
        grid=(n2,),
        in_specs=[
            pl.BlockSpec((R2, F), lambda i: (i, 0)),
            pl.BlockSpec((1, F), lambda i: (0, 0)),
            pl.BlockSpec((1, F), lambda i: (0, 0)),
            pl.BlockSpec((1, NB), lambda i: (0, 0)),
            pl.BlockSpec((NB, D), lambda i: (0, 0)),
            pl.BlockSpec((1, D), lambda i: (0, 0)),
        ],
        out_specs=pl.BlockSpec((R2, D), lambda i: (i, 0)),
        out_shape=jax.ShapeDtypeStruct((N, D), jnp.float32),
        compiler_params=pltpu.CompilerParams(
            dimension_semantics=("parallel",)),
    )(x, xmin, xmax, bins_ext, tdiff, b2)
    return out


# speedup vs baseline: 348.2246x; 3.8099x over previous
import functools

import jax
import jax.numpy as jnp
import numpy as np
from jax.experimental import pallas as pl
from jax.experimental.pallas import tpu as pltpu


def _minmax_body(x_ref, min_ref, max_ref, mn_s, mx_s, *, nsteps):
    i = pl.program_id(0)
    bmin = jnp.min(x_ref[...], axis=0, keepdims=True)
    bmax = jnp.max(x_ref[...], axis=0, keepdims=True)

    @pl.when(i == 0)
    def _():
        mn_s[...] = bmin
        mx_s[...] = bmax

    @pl.when(i > 0)
    def _():
        mn_s[...] = jnp.minimum(mn_s[...], bmin)
        mx_s[...] = jnp.maximum(mx_s[...], bmax)

    @pl.when(i == nsteps - 1)
    def _():
        min_ref[...] = mn_s[...]
        max_ref[...] = mx_s[...]


def _bucket_body(x_ref, mn_ref, mx_ref, bins_ref, td_ref, b2_ref, o_ref):
    mn = mn_ref[...]
    inv = 1.0 / (mx_ref[...] - mn + 1e-6)
    xn = (x_ref[...] - mn) * inv
    binsr = bins_ref[...]
    td = td_ref[...]
    nfeat = xn.shape[1]
    cnt = None
    for f in range(nfeat):
        col = xn[:, f:f + 1]
        c = (col <= binsr).astype(jnp.float32)
        cnt = c if cnt is None else cnt + c
    o_ref[...] = jnp.dot(cnt, td, preferred_element_type=jnp.float32,
                         precision=jax.lax.Precision.HIGHEST) + b2_ref[...]


def kernel(x, bin_logits, embed_table, W, b):
    N, F = x.shape
    NB = bin_logits.shape[0]
    D = W.shape[0]

    probs = jax.nn.softmax(bin_logits)
    bins = jnp.cumsum(probs)
    bins_ext = bins.at[NB - 1].set(jnp.inf).reshape(1, NB)
    table2 = embed_table @ W.T
    tdiff = jnp.concatenate([table2[:-1] - table2[1:], table2[-1:]], axis=0)
    b2 = (float(F) * b).reshape(1, D)

    R1 = 2000
    n1 = N // R1
    xmin, xmax = pl.pallas_call(
        functools.partial(_minmax_body, nsteps=n1),
        grid=(n1,),
        in_specs=[pl.BlockSpec((R1, F), lambda i: (i, 0))],
        out_specs=[pl.BlockSpec((1, F), lambda i: (0, 0)),
                   pl.BlockSpec((1, F), lambda i: (0, 0))],
        out_shape=[jax.ShapeDtypeStruct((1, F), jnp.float32),
                   jax.ShapeDtypeStruct((1, F), jnp.float32)],
        scratch_shapes=[pltpu.VMEM((1, F), jnp.float32),
                        pltpu.VMEM((1, F), jnp.float32)],
        compiler_params=pltpu.CompilerParams(
            dimension_semantics=("arbitrary",)),
    )(x)

    R2 = 1000
    n2 = N // R2
    out = pl.pallas_call(
        _bucket_body,
        grid=(n2,),
        in_specs=[
            pl.BlockSpec((R2, F), lambda i: (i, 0)),
            pl.BlockSpec((1, F), lambda i: (0, 0)),
            pl.BlockSpec((1, F), lambda i: (0, 0)),
            pl.BlockSpec((1, NB), lambda i: (0, 0)),
            pl.BlockSpec((NB, D), lambda i: (0, 0)),
            pl.BlockSpec((1, D), lambda i: (0, 0)),
        ],
        out_specs=pl.BlockSpec((R2, D), lambda i: (i, 0)),
        out_shape=jax.ShapeDtypeStruct((N, D), jnp.float32),
        compiler_params=pltpu.CompilerParams(
            dimension_semantics=("parallel",)),
    )(x, xmin, xmax, bins_ext, tdiff, b2)
    return out
